# Initial kernel scaffold; baseline (speedup 1.0000x reference)
import jax
import jax.numpy as jnp
from jax.experimental import pallas as pl


def kernel(x, support, W, b):
    raise NotImplementedError("write your pallas kernel here")


# speedup vs baseline: 4.3650x; 4.3650x over previous
import functools

import jax
import jax.numpy as jnp
from jax.experimental import pallas as pl
from jax.experimental.pallas import tpu as pltpu


def _fused_matmul_kernel(x_ref, B_ref, b_ref, o_ref, *, CV, TL):
    Cout, V = o_ref.shape[1], o_ref.shape[2]
    xb = x_ref[...].reshape(CV, TL).astype(jnp.bfloat16)
    acc = jnp.dot(B_ref[...], xb, preferred_element_type=jnp.float32)
    acc = acc.reshape(Cout, V, TL) + b_ref[...][:, :, None]
    o_ref[0] = acc.astype(o_ref.dtype)


def _fold_weights(support, W, C, V):
    S = support.shape[0]
    Cout, Ctot = W.shape[0], W.shape[1]
    order = (Ctot // C - 1) // S
    mats = [jnp.eye(V, dtype=jnp.float32)]
    for s in range(S):
        At = jnp.transpose(support[s]).astype(jnp.float32)
        Mk = jnp.eye(V, dtype=jnp.float32)
        for _ in range(order):
            Mk = jnp.dot(At, Mk)
            mats.append(Mk)
    Ms = jnp.stack(mats, 0)
    Wb = W.reshape(Cout, Ms.shape[0], C).astype(jnp.float32)
    B = jnp.einsum('obc,bvw->ovcw', Wb, Ms)
    return B.reshape(Cout * V, C * V)


def kernel(x, support, W, b):
    N, C, V, L = x.shape
    Cout = W.shape[0]
    CV = C * V

    B = _fold_weights(support, W, C, V).astype(jnp.bfloat16)
    b2 = b.reshape(Cout, 1).astype(jnp.float32)

    TL = 512 if (L % 512 == 0) else (256 if L % 256 == 0 else L)
    grid = (N, L // TL)

    flops = 2 * (Cout * V) * CV * N * L
    bytes_accessed = 4 * (N * C * V * L + N * Cout * V * L) + 2 * Cout * V * CV

    kernel_fn = functools.partial(_fused_matmul_kernel, CV=CV, TL=TL)
    out = pl.pallas_call(
        kernel_fn,
        out_shape=jax.ShapeDtypeStruct((N, Cout, V, L), x.dtype),
        grid=grid,
        in_specs=[
            pl.BlockSpec((1, C, V, TL), lambda n, i: (n, 0, 0, i)),
            pl.BlockSpec((Cout * V, CV), lambda n, i: (0, 0)),
            pl.BlockSpec((Cout, 1), lambda n, i: (0, 0)),
        ],
        out_specs=pl.BlockSpec((1, Cout, V, TL), lambda n, i: (n, 0, 0, i)),
        compiler_params=pltpu.CompilerParams(
            dimension_semantics=("parallel", "parallel")),
        cost_estimate=pl.CostEstimate(flops=int(flops), transcendentals=0,
                                      bytes_accessed=int(bytes_accessed)),
    )(x, B, b2)
    return out


# speedup vs baseline: 5.6285x; 1.2895x over previous
import functools

import jax
import jax.numpy as jnp
from jax.experimental import pallas as pl
from jax.experimental.pallas import tpu as pltpu


def _fused_matmul_kernel(x_ref, B_ref, b_ref, o_ref, *, CV, TL, BN):
    Cout, V = o_ref.shape[1], o_ref.shape[2]
    for j in range(BN):
        xb = x_ref[j].reshape(CV, TL).astype(jnp.bfloat16)
        acc = jnp.dot(B_ref[...], xb, preferred_element_type=jnp.float32)
        acc = acc.reshape(Cout, V, TL) + b_ref[...][:, :, None]
        o_ref[j] = acc.astype(o_ref.dtype)


def _fold_weights(support, W, C, V):
    S = support.shape[0]
    Cout, Ctot = W.shape[0], W.shape[1]
    order = (Ctot // C - 1) // S
    mats = [jnp.eye(V, dtype=jnp.float32)]
    for s in range(S):
        At = jnp.transpose(support[s]).astype(jnp.float32)
        Mk = jnp.eye(V, dtype=jnp.float32)
        for _ in range(order):
            Mk = jnp.dot(At, Mk)
            mats.append(Mk)
    Ms = jnp.stack(mats, 0)
    Wb = W.reshape(Cout, Ms.shape[0], C).astype(jnp.float32)
    B = jnp.einsum('obc,bvw->ovcw', Wb, Ms)
    return B.reshape(Cout * V, C * V)


def kernel(x, support, W, b):
    N, C, V, L = x.shape
    Cout = W.shape[0]
    CV = C * V

    B = _fold_weights(support, W, C, V).astype(jnp.bfloat16)
    b2 = b.reshape(Cout, 1).astype(jnp.float32)

    TL = 512 if (L % 512 == 0) else (256 if L % 256 == 0 else L)
    BN = 4 if (N % 4 == 0 and TL == L) else 1
    grid = (N // BN, L // TL)

    flops = 2 * (Cout * V) * CV * N * L
    bytes_accessed = 4 * (N * C * V * L + N * Cout * V * L) + 2 * Cout * V * CV

    kernel_fn = functools.partial(_fused_matmul_kernel, CV=CV, TL=TL, BN=BN)
    out = pl.pallas_call(
        kernel_fn,
        out_shape=jax.ShapeDtypeStruct((N, Cout, V, L), x.dtype),
        grid=grid,
        in_specs=[
            pl.BlockSpec((BN, C, V, TL), lambda n, i: (n, 0, 0, i)),
            pl.BlockSpec((Cout * V, CV), lambda n, i: (0, 0)),
            pl.BlockSpec((Cout, 1), lambda n, i: (0, 0)),
        ],
        out_specs=pl.BlockSpec((BN, Cout, V, TL), lambda n, i: (n, 0, 0, i)),
        compiler_params=pltpu.CompilerParams(
            dimension_semantics=("arbitrary", "arbitrary")),
        cost_estimate=pl.CostEstimate(flops=int(flops), transcendentals=0,
                                      bytes_accessed=int(bytes_accessed)),
    )(x, B, b2)
    return out


# speedup vs baseline: 5.6843x; 1.0099x over previous
import functools

import jax
import jax.numpy as jnp
from jax.experimental import pallas as pl
from jax.experimental.pallas import tpu as pltpu


def _fused_matmul_kernel(x_ref, B_ref, b_ref, o_ref, *, CV, TL, BN):
    Cout, V = o_ref.shape[1], o_ref.shape[2]
    for j in range(BN):
        xb = x_ref[j].reshape(CV, TL).astype(jnp.bfloat16)
        acc = jnp.dot(B_ref[...], xb, preferred_element_type=jnp.float32)
        acc = acc.reshape(Cout, V, TL) + b_ref[...][:, :, None]
        o_ref[j] = acc.astype(o_ref.dtype)


def _fold_weights(support, W, C, V):
    S = support.shape[0]
    Cout, Ctot = W.shape[0], W.shape[1]
    order = (Ctot // C - 1) // S
    mats = [jnp.eye(V, dtype=jnp.float32)]
    for s in range(S):
        At = jnp.transpose(support[s]).astype(jnp.float32)
        Mk = jnp.eye(V, dtype=jnp.float32)
        for _ in range(order):
            Mk = jnp.dot(At, Mk)
            mats.append(Mk)
    Ms = jnp.stack(mats, 0)
    Wb = W.reshape(Cout, Ms.shape[0], C).astype(jnp.float32)
    B = jnp.einsum('obc,bvw->ovcw', Wb, Ms)
    return B.reshape(Cout * V, C * V)


def kernel(x, support, W, b):
    N, C, V, L = x.shape
    Cout = W.shape[0]
    CV = C * V

    B = _fold_weights(support, W, C, V).astype(jnp.bfloat16)
    b2 = b.reshape(Cout, 1).astype(jnp.float32)

    TL = 512 if (L % 512 == 0) else (256 if L % 256 == 0 else L)
    BN = 8 if (N % 8 == 0 and TL == L) else 1
    grid = (N // BN, L // TL)

    flops = 2 * (Cout * V) * CV * N * L
    bytes_accessed = 4 * (N * C * V * L + N * Cout * V * L) + 2 * Cout * V * CV

    kernel_fn = functools.partial(_fused_matmul_kernel, CV=CV, TL=TL, BN=BN)
    out = pl.pallas_call(
        kernel_fn,
        out_shape=jax.ShapeDtypeStruct((N, Cout, V, L), x.dtype),
        grid=grid,
        in_specs=[
            pl.BlockSpec((BN, C, V, TL), lambda n, i: (n, 0, 0, i)),
            pl.BlockSpec((Cout * V, CV), lambda n, i: (0, 0)),
            pl.BlockSpec((Cout, 1), lambda n, i: (0, 0)),
        ],
        out_specs=pl.BlockSpec((BN, Cout, V, TL), lambda n, i: (n, 0, 0, i)),
        compiler_params=pltpu.CompilerParams(
            dimension_semantics=("arbitrary", "arbitrary")),
        cost_estimate=pl.CostEstimate(flops=int(flops), transcendentals=0,
                                      bytes_accessed=int(bytes_accessed)),
    )(x, B, b2)
    return out
